# baseline (speedup 1.0000x reference)
import functools

import jax
import jax.numpy as jnp
from jax import lax
from jax.experimental import pallas as pl
from jax.experimental.pallas import tpu as pltpu
from jax.experimental.pallas import tpu_sc as plsc

_TN = 2048


def _tc_index_body(xh_ref, xn_ref, ke_ref, kn_ref, idx_ref):
    tn = xh_ref.shape[1]
    c = ke_ref.shape[1]
    cs = 512
    xb = xh_ref[0]
    xn = xn_ref[0]
    parts = []
    m = None
    for k in range(c // cs):
        dot = lax.dot_general(xb, ke_ref[0, pl.ds(k * cs, cs), :],
                              (((1,), (1,)), ((), ())),
                              preferred_element_type=jnp.float32)
        d2 = (xn - 2.0 * dot) + kn_ref[0, :, pl.ds(k * cs, cs)]
        s = jnp.sqrt(jnp.maximum(d2, 0.0))
        parts.append(s)
        mk = jnp.min(s, axis=1, keepdims=True)
        m = mk if m is None else jnp.minimum(m, mk)
    idx = None
    for k, s in enumerate(parts):
        iota = lax.broadcasted_iota(jnp.int32, (tn, cs), 1) + k * cs
        sel = jnp.where(s == m, iota, c)
        ik = jnp.min(sel, axis=1, keepdims=True)
        idx = ik if idx is None else jnp.minimum(idx, ik)
    idx_ref[0] = idx.reshape(tn // 128, 128)


def _nearest_code_indices(xh, xn, key_embed, kn):
    h, n, d = xh.shape
    c = key_embed.shape[1]
    tn = _TN
    return pl.pallas_call(
        _tc_index_body,
        grid=(h, n // tn),
        in_specs=[
            pl.BlockSpec((1, tn, d), lambda hh, i: (hh, i, 0)),
            pl.BlockSpec((1, tn, 1), lambda hh, i: (hh, i, 0)),
            pl.BlockSpec((1, c, d), lambda hh, i: (hh, 0, 0)),
            pl.BlockSpec((1, 1, c), lambda hh, i: (hh, 0, 0)),
        ],
        out_specs=pl.BlockSpec((1, tn // 128, 128), lambda hh, i: (hh, i, 0)),
        out_shape=jax.ShapeDtypeStruct((h, n // 128, 128), jnp.int32),
    )(xh, xn, key_embed, kn)


def _sc_gather_values(values128, idx128, h, n, dv):
    info = plsc.get_sparse_core_info()
    nw = info.num_cores * info.num_subcores
    hpw = h // nw
    tab_rows = values128.shape[1]
    idx_rows = idx128.shape[1]
    out_rows = dv * n // 128
    ngroups = n // 16
    mesh = plsc.VectorSubcoreMesh(core_axis_name="c", subcore_axis_name="s")

    def body(tab_hbm, idx_hbm, out_hbm, tab_v, idx_v, out_v):
        wid = lax.axis_index("s") * info.num_cores + lax.axis_index("c")
        for hloc in range(hpw):
            head = wid * hpw + hloc
            pltpu.sync_copy(tab_hbm.at[head], tab_v)
            pltpu.sync_copy(idx_hbm.at[head], idx_v)

            def group(g, carry):
                ids = idx_v[g >> 3, pl.ds((g & 7) * 16, 16)]
                w0 = ids * dv
                orow0 = g >> 3
                ocol = (g & 7) * 16
                for j in range(dv):
                    w = w0 + j
                    v = plsc.load_gather(tab_v, [w >> 7, w & 127])
                    out_v[j * idx_rows + orow0, pl.ds(ocol, 16)] = v
                return carry

            lax.fori_loop(0, ngroups, group, 0)
            pltpu.sync_copy(out_v, out_hbm.at[head])

    return pl.kernel(
        body,
        out_type=jax.ShapeDtypeStruct((h, out_rows, 128), jnp.float32),
        mesh=mesh,
        compiler_params=pltpu.CompilerParams(use_tc_tiling_on_sc=False,
                                             needs_layout_passes=False),
        scratch_types=[
            pltpu.VMEM((tab_rows, 128), jnp.float32),
            pltpu.VMEM((idx_rows, 128), jnp.int32),
            pltpu.VMEM((out_rows, 128), jnp.float32),
        ],
    )(values128, idx128)


def kernel(x, mask, key_embed, values, key_optim):
    x = x.astype(jnp.float32)
    b, t, dim = x.shape
    h, c, d = key_embed.shape
    dv = values.shape[-1]
    n = b * t

    xh = jnp.transpose(x.reshape(b, t, h, d), (2, 0, 1, 3)).reshape(h, n, d)
    xn = jnp.sum(xh ** 2, axis=-1, keepdims=True)
    kn = jnp.sum(key_embed ** 2, axis=-1)[:, None, :]

    idx128 = _nearest_code_indices(xh, xn, key_embed, kn)

    values128 = values.reshape(h, c * dv // 128, 128)
    out = _sc_gather_values(values128, idx128, h, n, dv)
    out = jnp.transpose(out.reshape(h, dv, n), (2, 0, 1)).reshape(b, t, h * dv)
    return out


# speedup vs baseline: 1.0153x; 1.0153x over previous
import functools

import jax
import jax.numpy as jnp
from jax import lax
from jax.experimental import pallas as pl
from jax.experimental.pallas import tpu as pltpu
from jax.experimental.pallas import tpu_sc as plsc

_TN = 2048


def _tc_index_body(xh_ref, xn_ref, ke_ref, kn_ref, idx_ref):
    tn = xh_ref.shape[1]
    c = ke_ref.shape[1]
    dot = lax.dot_general(xh_ref[0], ke_ref[0], (((1,), (1,)), ((), ())),
                          preferred_element_type=jnp.float32)
    d2 = (xn_ref[0] - 2.0 * dot) + kn_ref[0]
    s = jnp.sqrt(jnp.maximum(d2, 0.0))
    m = jnp.min(s, axis=1, keepdims=True)
    iota = lax.broadcasted_iota(jnp.int32, (tn, c), 1)
    sel = jnp.where(s == m, iota, c)
    idx = jnp.min(sel, axis=1, keepdims=True)
    idx_ref[0] = idx.reshape(tn // 128, 128)


def _nearest_code_indices(xh, xn, key_embed, kn):
    h, n, d = xh.shape
    c = key_embed.shape[1]
    tn = _TN
    return pl.pallas_call(
        _tc_index_body,
        grid=(h, n // tn),
        in_specs=[
            pl.BlockSpec((1, tn, d), lambda hh, i: (hh, i, 0)),
            pl.BlockSpec((1, tn, 1), lambda hh, i: (hh, i, 0)),
            pl.BlockSpec((1, c, d), lambda hh, i: (hh, 0, 0)),
            pl.BlockSpec((1, 1, c), lambda hh, i: (hh, 0, 0)),
        ],
        out_specs=pl.BlockSpec((1, tn // 128, 128), lambda hh, i: (hh, i, 0)),
        out_shape=jax.ShapeDtypeStruct((h, n // 128, 128), jnp.int32),
    )(xh, xn, key_embed, kn)


def _sc_gather_values(values128, idx128, h, n, dv):
    info = plsc.get_sparse_core_info()
    nw = info.num_cores * info.num_subcores
    hpw = h // nw
    tab_rows = values128.shape[1]
    idx_rows = idx128.shape[1]
    out_rows = dv * n // 128
    ngroups = n // 16
    mesh = plsc.VectorSubcoreMesh(core_axis_name="c", subcore_axis_name="s")

    def body(tab_hbm, idx_hbm, out_hbm, tab_v, idx_v, out_v):
        wid = lax.axis_index("s") * info.num_cores + lax.axis_index("c")
        for hloc in range(hpw):
            head = wid * hpw + hloc
            pltpu.sync_copy(tab_hbm.at[head], tab_v)
            pltpu.sync_copy(idx_hbm.at[head], idx_v)

            def group(g, carry):
                ids = idx_v[g >> 3, pl.ds((g & 7) * 16, 16)]
                w0 = ids * dv
                orow0 = g >> 3
                ocol = (g & 7) * 16
                for j in range(dv):
                    w = w0 + j
                    v = plsc.load_gather(tab_v, [w >> 7, w & 127])
                    out_v[j * idx_rows + orow0, pl.ds(ocol, 16)] = v
                return carry

            lax.fori_loop(0, ngroups, group, 0)
            pltpu.sync_copy(out_v, out_hbm.at[head])

    return pl.kernel(
        body,
        out_type=jax.ShapeDtypeStruct((h, out_rows, 128), jnp.float32),
        mesh=mesh,
        compiler_params=pltpu.CompilerParams(use_tc_tiling_on_sc=False,
                                             needs_layout_passes=False),
        scratch_types=[
            pltpu.VMEM((tab_rows, 128), jnp.float32),
            pltpu.VMEM((idx_rows, 128), jnp.int32),
            pltpu.VMEM((out_rows, 128), jnp.float32),
        ],
    )(values128, idx128)


def kernel(x, mask, key_embed, values, key_optim):
    x = x.astype(jnp.float32)
    b, t, dim = x.shape
    h, c, d = key_embed.shape
    dv = values.shape[-1]
    n = b * t

    xh = jnp.transpose(x.reshape(b, t, h, d), (2, 0, 1, 3)).reshape(h, n, d)
    xn = jnp.sum(xh ** 2, axis=-1, keepdims=True)
    kn = jnp.sum(key_embed ** 2, axis=-1)[:, None, :]

    idx128 = _nearest_code_indices(xh, xn, key_embed, kn)

    values128 = values.reshape(h, c * dv // 128, 128)
    out = _sc_gather_values(values128, idx128, h, n, dv)
    out = jnp.transpose(out.reshape(h, dv, n), (2, 0, 1)).reshape(b, t, h * dv)
    return out
